# Initial kernel scaffold; baseline (speedup 1.0000x reference)
import jax
import jax.numpy as jnp
from jax.experimental import pallas as pl


def kernel(x, edge_index_rel0, edge_index_rel1, batch, W, Wih, Whh, bih, bhh, clf_w, clf_b):
    raise NotImplementedError("write your pallas kernel here")


# speedup vs baseline: 4.5499x; 4.5499x over previous
import functools

import jax
import jax.numpy as jnp
from jax import lax
from jax.experimental import pallas as pl
from jax.experimental.pallas import tpu as pltpu
from jax.experimental.pallas import tpu_sc as plsc

N = 10000
E = 320000
H = 128
LL = 3
RR = 2
KK = 3
G = 64

NC = 2
NS = 16
NW = NC * NS
EPW = E // NW
CH = 80
NCH = EPW // CH
NP = 10240
RPT = NP // NS
ZR = CH

PW = 25
NPW = N // PW
PCH = NPW // CH

def _sc_mesh():
    return plsc.VectorSubcoreMesh(core_axis_name="c", subcore_axis_name="s",
                                  num_cores=NC, num_subcores=NS)


_SC_SCATTER_KW = dict(
    out_type=jax.ShapeDtypeStruct((NC * NP, H), jnp.float32),
    scratch_types=[
        pltpu.VMEM((NCH, CH), jnp.int32),
        pltpu.VMEM((NCH, CH), jnp.int32),
        pltpu.VMEM((CH, H), jnp.float32),
        pltpu.VMEM_SHARED((NP, H), jnp.float32),
        pltpu.SemaphoreType.DMA,
    ],
)


def _sc_scatter_body(h_hbm, src_hbm, dst_hbm, zero_hbm, out_hbm,
                     src_v, dst_v, rows_v, agg_sh, sem):
    cid = lax.axis_index("c")
    sid = lax.axis_index("s")
    wid = cid * NS + sid
    pltpu.sync_copy(zero_hbm, rows_v)
    base_r = sid * RPT
    for i in range(RPT // ZR):
        pltpu.sync_copy(rows_v, agg_sh.at[pl.ds(base_r + i * ZR, ZR)])
    plsc.subcore_barrier()
    pltpu.sync_copy(src_hbm.at[wid], src_v)
    pltpu.sync_copy(dst_hbm.at[wid], dst_v)

    def body(j, carry):
        pltpu.async_copy(h_hbm.at[src_v.at[j]], rows_v, sem).wait()
        pltpu.sync_copy(rows_v, agg_sh.at[dst_v.at[j]], add=True)
        return carry

    lax.fori_loop(0, NCH, body, 0)
    plsc.subcore_barrier()
    for i in range(RPT // ZR):
        pltpu.sync_copy(agg_sh.at[pl.ds(base_r + i * ZR, ZR)], rows_v)
        pltpu.sync_copy(rows_v, out_hbm.at[pl.ds(cid * NP + base_r + i * ZR, ZR)])


_SC_POOL_KW = dict(
    out_type=(jax.ShapeDtypeStruct((NC * G, H), jnp.float32),
              jax.ShapeDtypeStruct((NC * G, 16), jnp.float32)),
    scratch_types=[
        pltpu.VMEM((PCH, CH), jnp.int32),
        pltpu.VMEM((CH, H), jnp.float32),
        pltpu.VMEM((CH, 16), jnp.float32),
        pltpu.VMEM((G, H), jnp.float32),
        pltpu.VMEM((G, 16), jnp.float32),
        pltpu.VMEM_SHARED((G, H), jnp.float32),
        pltpu.VMEM_SHARED((G, 16), jnp.float32),
        pltpu.SemaphoreType.DMA,
    ],
)


def _sc_pool_body(h_hbm, b_hbm, zs_hbm, zc_hbm, ones_hbm, sums_out, cnts_out,
                  bidx_v, rows_v, ones_v, sbuf, cbuf, sums_sh, cnts_sh, sem):
    cid = lax.axis_index("c")
    sid = lax.axis_index("s")
    wid = cid * NS + sid

    @pl.when(sid == 0)
    def _init():
        pltpu.sync_copy(zs_hbm, sbuf)
        pltpu.sync_copy(sbuf, sums_sh)
        pltpu.sync_copy(zc_hbm, cbuf)
        pltpu.sync_copy(cbuf, cnts_sh)

    pltpu.sync_copy(ones_hbm, ones_v)
    plsc.subcore_barrier()

    @pl.when(wid < PW)
    def _scatter():
        base = wid * NPW
        pltpu.sync_copy(b_hbm.at[wid], bidx_v)

        def body(j, carry):
            pltpu.async_copy(h_hbm.at[pl.ds(base + j * CH, CH)], rows_v,
                             sem).wait()
            pltpu.sync_copy(rows_v, sums_sh.at[bidx_v.at[j]], add=True)
            pltpu.sync_copy(ones_v, cnts_sh.at[bidx_v.at[j]], add=True)
            return carry

        lax.fori_loop(0, PCH, body, 0)

    plsc.subcore_barrier()

    @pl.when(sid == 0)
    def _writeout():
        pltpu.sync_copy(sums_sh, sbuf)
        pltpu.sync_copy(sbuf, sums_out.at[pl.ds(cid * G, G)])
        pltpu.sync_copy(cnts_sh, cbuf)
        pltpu.sync_copy(cbuf, cnts_out.at[pl.ds(cid * G, G)])


_sc_lazy = {}


def _sc_kernels():
    if not _sc_lazy:
        mesh = _sc_mesh()
        _sc_lazy["scatter"] = pl.kernel(_sc_scatter_body, mesh=mesh,
                                        **_SC_SCATTER_KW)
        _sc_lazy["pool"] = pl.kernel(_sc_pool_body, mesh=mesh, **_SC_POOL_KW)
    return _sc_lazy["scatter"], _sc_lazy["pool"]


BR = 2000


def _sigmoid(u):
    return 1.0 / (1.0 + jnp.exp(-u))


def _tanh(u):
    e = jnp.exp(-2.0 * u)
    return (1.0 - e) / (1.0 + e)


def _gru_body(a0, a1, h, wk, wih, whh, bi, bh, o):
    aggh = a0[...] + a1[...]
    agg = jnp.dot(aggh, wk[...], preferred_element_type=jnp.float32,
                 precision=lax.Precision.HIGHEST)
    gi = jnp.dot(agg, wih[...], preferred_element_type=jnp.float32,
                 precision=lax.Precision.HIGHEST) + bi[...]
    gh = jnp.dot(h[...], whh[...], preferred_element_type=jnp.float32,
                 precision=lax.Precision.HIGHEST) + bh[...]
    hv = h[...]
    r = _sigmoid(gi[:, :H] + gh[:, :H])
    z = _sigmoid(gi[:, H:2 * H] + gh[:, H:2 * H])
    n = _tanh(gi[:, 2 * H:] + r * gh[:, 2 * H:])
    o[...] = n + z * (hv - n)


def _gru(a0, a1, h, wk, wihT, whhT, bi, bh):
    row = pl.BlockSpec((BR, H), lambda i: (i, 0))

    def full(r, c):
        return pl.BlockSpec((r, c), lambda i: (0, 0))

    return pl.pallas_call(
        _gru_body,
        grid=(N // BR,),
        in_specs=[row, row, row, full(H, H), full(H, 3 * H), full(H, 3 * H),
                  full(1, 3 * H), full(1, 3 * H)],
        out_specs=row,
        out_shape=jax.ShapeDtypeStruct((N, H), jnp.float32),
    )(a0, a1, h, wk, wihT, whhT, bi, bh)


def _relu_add_body(a, b, o):
    o[...] = jnp.maximum(a[...] + b[...], 0.0)


def _relu_add(a, b):
    row = pl.BlockSpec((BR, H), lambda i: (i, 0))
    return pl.pallas_call(
        _relu_add_body,
        grid=(N // BR,),
        in_specs=[row, row],
        out_specs=row,
        out_shape=jax.ShapeDtypeStruct((N, H), jnp.float32),
    )(a, b)


def _final_body(s0, s1, c0, c1, cw, cb, o):
    s = s0[...] + s1[...]
    c = c0[...][:, :1] + c1[...][:, :1]
    rep = s / jnp.maximum(c, 1.0)
    logit = jnp.dot(rep, cw[...], preferred_element_type=jnp.float32,
                 precision=lax.Precision.HIGHEST) + cb[...]
    o[...] = _sigmoid(logit)


def _final(s0, s1, c0, c1, cw, cb):
    def full(r, c):
        return pl.BlockSpec((r, c), lambda: (0, 0))

    return pl.pallas_call(
        _final_body,
        in_specs=[full(G, H), full(G, H), full(G, 16), full(G, 16),
                  full(H, 1), full(1, 1)],
        out_specs=full(G, 1),
        out_shape=jax.ShapeDtypeStruct((G, 1), jnp.float32),
    )(s0, s1, c0, c1, cw, cb)


def kernel(x, edge_index_rel0, edge_index_rel1, batch, W, Wih, Whh, bih, bhh,
           clf_w, clf_b):
    src = [edge_index_rel0[0].reshape(NW, NCH, CH),
           edge_index_rel1[0].reshape(NW, NCH, CH)]
    dst = [edge_index_rel0[1].reshape(NW, NCH, CH),
           edge_index_rel1[1].reshape(NW, NCH, CH)]
    WihT = jnp.swapaxes(Wih, -1, -2)
    WhhT = jnp.swapaxes(Whh, -1, -2)
    bi2 = bih.reshape(LL, RR, 1, 3 * H)
    bh2 = bhh.reshape(LL, RR, 1, 3 * H)
    zero_rows = jnp.zeros((CH, H), jnp.float32)

    _sc_scatter, _sc_pool = _sc_kernels()
    h = x
    for l in range(LL):
        hs = []
        for r in range(RR):
            hr = h
            for k in range(KK):
                aggp = _sc_scatter(hr, src[r], dst[r], zero_rows)
                hr = _gru(aggp[:N], aggp[NP:NP + N], hr, W[l, r, k],
                          WihT[l, r], WhhT[l, r], bi2[l, r], bh2[l, r])
            hs.append(hr)
        h = _relu_add(hs[0], hs[1])

    b2 = jnp.zeros((NW, PCH, CH), jnp.int32).at[:PW].set(
        batch.reshape(PW, PCH, CH))
    zs = jnp.zeros((G, H), jnp.float32)
    zc = jnp.zeros((G, 16), jnp.float32)
    ones = jnp.ones((CH, 16), jnp.float32)
    sums, cnts = _sc_pool(h, b2, zs, zc, ones)
    out = _final(sums[:G], sums[G:], cnts[:G], cnts[G:],
                 clf_w.reshape(H, 1), clf_b.reshape(1, 1))
    return out.reshape(G)
